# baseline (speedup 1.0000x reference)
import functools
import math

import jax
import jax.numpy as jnp
from jax import lax
from jax.experimental import pallas as pl
from jax.experimental.pallas import tpu as pltpu
from jax.experimental.pallas import tpu_sc as plsc

V = 1000000
D = 64
SCALE = math.sqrt(D)
L = 16
NW = 32

FULL_COLS = V // 128
REM = V - FULL_COLS * 128


def _make_stage1():
    mesh = plsc.VectorSubcoreMesh(core_axis_name="c", subcore_axis_name="s")

    @functools.partial(
        pl.kernel,
        mesh=mesh,
        out_type=jax.ShapeDtypeStruct((V // 2, 128), jnp.float32),
        scratch_types=[
            pltpu.VMEM((2, 64, 256), jnp.float32),
            pltpu.VMEM((2, 128, 128), jnp.float32),
            pltpu.VMEM((64, 64), jnp.float32),
            pltpu.VMEM((32, 128), jnp.float32),
            pltpu.VMEM((L, L), jnp.int32),
            pltpu.SemaphoreType.DMA((2,)),
            pltpu.SemaphoreType.DMA((2,)),
        ],
        compiler_params=pltpu.CompilerParams(
            use_tc_tiling_on_sc=True, needs_layout_passes=False
        ),
    )
    def stage1(tabT_hbm, z_hbm, inb, outb, pin, pout, rottab, rsem, wsem):
        wid = lax.axis_index("s") * 2 + lax.axis_index("c")
        start = 244 * wid + 2 * jnp.minimum(wid, 2)
        n_pairs = jnp.where(wid < 2, 123, 122)

        iota = lax.iota(jnp.int32, L)
        for k in range(L):
            rottab[k, pl.ds(0, L)] = (iota + k) & 15

        def transpose_block(src, dst, n_m, m_base):
            @plsc.parallel_loop(0, n_m, L, unroll=2)
            def _t(m0):
                bc_m0 = jnp.broadcast_to(m0, (L,))
                for c0 in range(0, D, L):
                    crow = c0 + iota
                    for k in range(L):
                        colv = bc_m0 + rottab[k, pl.ds(0, L)]
                        orow = m_base + (colv >> 1)
                        ocol = ((colv & 1) << 6) + crow
                        v = plsc.load_gather(src, [crow, colv])
                        plsc.store_scatter(dst, [orow, ocol], v)

        def fire_read(i, b):
            tc = start + 2 * i
            pltpu.async_copy(
                tabT_hbm.at[pl.ds(0, D), pl.ds(tc * 128, 256)],
                inb.at[b],
                rsem.at[b],
            )

        fire_read(0, 0)

        @pl.loop(0, 124, step=2)
        def outer(i0):
            for b in range(2):
                i = i0 + b
                b1 = (b + 1) % 2

                @pl.when(i < n_pairs)
                def _():
                    @pl.when(i + 1 < n_pairs)
                    def _():
                        fire_read(i + 1, b1)

                    pltpu.make_async_copy(
                        tabT_hbm.at[pl.ds(0, D), pl.ds(0, 256)],
                        inb.at[b],
                        rsem.at[b],
                    ).wait()

                    @pl.when(i >= 2)
                    def _():
                        pltpu.make_async_copy(
                            outb.at[b],
                            z_hbm.at[pl.ds(0, 128)],
                            wsem.at[b],
                        ).wait()

                    transpose_block(inb.at[b], outb.at[b], 256, 0)

                    tc = start + 2 * i
                    pltpu.async_copy(
                        outb.at[b],
                        z_hbm.at[pl.ds(tc * 64, 128)],
                        wsem.at[b],
                    )

        @pl.when(wid == NW - 1)
        def _():
            pltpu.sync_copy(
                tabT_hbm.at[pl.ds(0, D), pl.ds(FULL_COLS * 128, REM)], pin
            )

            transpose_block(pin, pout, REM, 0)

            pltpu.sync_copy(pout, z_hbm.at[pl.ds(FULL_COLS * 64, REM // 2)])

        for b in range(2):
            @pl.when(n_pairs >= 2 - b)
            def _():
                pltpu.make_async_copy(
                    outb.at[b], z_hbm.at[pl.ds(0, 128)], wsem.at[b]
                ).wait()

    return stage1


def _make_stage2(NB_T):
    mesh = plsc.VectorSubcoreMesh(core_axis_name="c", subcore_axis_name="s")

    @functools.partial(
        pl.kernel,
        mesh=mesh,
        out_type=jax.ShapeDtypeStruct((NB_T, 8, 32, 8, 128), jnp.float32),
        scratch_types=[
            pltpu.VMEM((NB_T, 128), jnp.int32),
            pltpu.VMEM((2, 128, D), jnp.float32),
            pltpu.VMEM((2, 8, 8, 128), jnp.float32),
            pltpu.VMEM((128, L), jnp.int32),
            pltpu.SemaphoreType.DMA((2,)),
            pltpu.SemaphoreType.DMA((2,)),
        ],
        compiler_params=pltpu.CompilerParams(
            use_tc_tiling_on_sc=False, needs_layout_passes=False
        ),
    )
    def stage2(xT_hbm, z_hbm, o5_hbm, idxT, G, O, rowtab, gsem, wsem):
        wid = lax.axis_index("s") * 2 + lax.axis_index("c")
        pltpu.sync_copy(
            xT_hbm.at[pl.ds(0, NB_T), pl.ds(wid * 128, 128)], idxT
        )

        iota = lax.iota(jnp.int32, L)
        for g in range(8):
            for k in range(L):
                rowtab[g * L + k, pl.ds(0, L)] = g * L + ((iota + k) & 15)

        def fire_gather(t, b):
            pltpu.async_copy(z_hbm.at[idxT.at[t]], G.at[b], gsem.at[b])

        fire_gather(0, 0)

        @pl.loop(0, NB_T, step=2)
        def outer(t0):
            for b in range(2):
                t = t0 + b
                b1 = (b + 1) % 2

                @pl.when(t + 1 < NB_T)
                def _():
                    fire_gather(t + 1, b1)

                pltpu.make_async_copy(
                    z_hbm.at[pl.ds(0, 128)], G.at[b], gsem.at[b]
                ).wait()

                @pl.when(t >= 2)
                def _():
                    pltpu.make_async_copy(
                        O.at[b],
                        o5_hbm.at[0, pl.ds(0, 8), 0],
                        wsem.at[b],
                    ).wait()

                @plsc.parallel_loop(0, 128, L, unroll=2)
                def _j(rid0):
                    for c0 in range(0, D, L):
                        colv = c0 + iota
                        c8v = colv >> 3
                        c1v = colv & 7
                        for k in range(L):
                            rowv = rowtab[rid0 + k, pl.ds(0, L)]
                            v = plsc.load_gather(G.at[b], [rowv, colv])
                            plsc.store_scatter(
                                O.at[b], [c8v, c1v, rowv], v * SCALE
                            )

                pltpu.async_copy(
                    O.at[b], o5_hbm.at[t, pl.ds(0, 8), wid], wsem.at[b]
                )

        for b in range(2):
            pltpu.make_async_copy(
                O.at[b], o5_hbm.at[0, pl.ds(0, 8), 0], wsem.at[b]
            ).wait()

    return stage2


def kernel(x, table):
    nb, nt = x.shape
    z = _make_stage1()(table.T)
    o5 = _make_stage2(nt)(x.T, jnp.reshape(z, (V, D)))
    return jnp.transpose(o5, (2, 4, 0, 1, 3)).reshape(nb, nt, D)


# speedup vs baseline: 2.0595x; 2.0595x over previous
import functools
import math

import jax
import jax.numpy as jnp
from jax import lax
from jax.experimental import pallas as pl
from jax.experimental.pallas import tpu as pltpu
from jax.experimental.pallas import tpu_sc as plsc

V = 1000000
D = 64
SCALE = math.sqrt(D)
L = 16
NW = 32

FULL_COLS = V // 128
REM = V - FULL_COLS * 128


def _make_stage1():
    mesh = plsc.VectorSubcoreMesh(core_axis_name="c", subcore_axis_name="s")

    @functools.partial(
        pl.kernel,
        mesh=mesh,
        out_type=jax.ShapeDtypeStruct((V // 2, 128), jnp.float32),
        scratch_types=[
            pltpu.VMEM((2, 64, 256), jnp.float32),
            pltpu.VMEM((2, 128, 128), jnp.float32),
            pltpu.VMEM((64, 64), jnp.float32),
            pltpu.VMEM((32, 128), jnp.float32),
            pltpu.SemaphoreType.DMA((2,)),
            pltpu.SemaphoreType.DMA((2,)),
        ],
        compiler_params=pltpu.CompilerParams(
            use_tc_tiling_on_sc=True, needs_layout_passes=False
        ),
    )
    def stage1(tabT_hbm, z_hbm, inb, outb, pin, pout, rsem, wsem):
        wid = lax.axis_index("s") * 2 + lax.axis_index("c")
        start = 244 * wid + 2 * jnp.minimum(wid, 2)
        n_pairs = jnp.where(wid < 2, 123, 122)

        iota = lax.iota(jnp.int32, L)
        rot = [(iota + k) & 15 for k in range(L)]

        def transpose_block(src, dst, n_m, m_base):
            @plsc.parallel_loop(0, n_m, L, unroll=2)
            def _t(m0):
                bc_m0 = jnp.broadcast_to(m0, (L,))
                for c0 in range(0, D, L):
                    crow = c0 + iota
                    for k in range(L):
                        colv = bc_m0 + rot[k]
                        orow = m_base + (colv >> 1)
                        ocol = ((colv & 1) << 6) + crow
                        v = plsc.load_gather(src, [crow, colv])
                        plsc.store_scatter(dst, [orow, ocol], v)

        def fire_read(i, b):
            tc = start + 2 * i
            pltpu.async_copy(
                tabT_hbm.at[pl.ds(0, D), pl.ds(tc * 128, 256)],
                inb.at[b],
                rsem.at[b],
            )

        fire_read(0, 0)

        @pl.loop(0, 124, step=2)
        def outer(i0):
            for b in range(2):
                i = i0 + b
                b1 = (b + 1) % 2

                @pl.when(i < n_pairs)
                def _():
                    @pl.when(i + 1 < n_pairs)
                    def _():
                        fire_read(i + 1, b1)

                    pltpu.make_async_copy(
                        tabT_hbm.at[pl.ds(0, D), pl.ds(0, 256)],
                        inb.at[b],
                        rsem.at[b],
                    ).wait()

                    @pl.when(i >= 2)
                    def _():
                        pltpu.make_async_copy(
                            outb.at[b],
                            z_hbm.at[pl.ds(0, 128)],
                            wsem.at[b],
                        ).wait()

                    transpose_block(inb.at[b], outb.at[b], 256, 0)

                    tc = start + 2 * i
                    pltpu.async_copy(
                        outb.at[b],
                        z_hbm.at[pl.ds(tc * 64, 128)],
                        wsem.at[b],
                    )

        @pl.when(wid == NW - 1)
        def _():
            pltpu.sync_copy(
                tabT_hbm.at[pl.ds(0, D), pl.ds(FULL_COLS * 128, REM)], pin
            )

            transpose_block(pin, pout, REM, 0)

            pltpu.sync_copy(pout, z_hbm.at[pl.ds(FULL_COLS * 64, REM // 2)])

        for b in range(2):
            @pl.when(n_pairs >= 2 - b)
            def _():
                pltpu.make_async_copy(
                    outb.at[b], z_hbm.at[pl.ds(0, 128)], wsem.at[b]
                ).wait()

    return stage1


def _make_stage2(NB_T):
    mesh = plsc.VectorSubcoreMesh(core_axis_name="c", subcore_axis_name="s")

    @functools.partial(
        pl.kernel,
        mesh=mesh,
        out_type=jax.ShapeDtypeStruct((NB_T, 8, 32, 8, 128), jnp.float32),
        scratch_types=[
            pltpu.VMEM((NB_T, 128), jnp.int32),
            pltpu.VMEM((2, 128, D), jnp.float32),
            pltpu.VMEM((2, 8, 8, 128), jnp.float32),
            pltpu.SemaphoreType.DMA((2,)),
            pltpu.SemaphoreType.DMA((2,)),
        ],
        compiler_params=pltpu.CompilerParams(
            use_tc_tiling_on_sc=False, needs_layout_passes=False
        ),
    )
    def stage2(xT_hbm, z_hbm, o5_hbm, idxT, G, O, gsem, wsem):
        wid = lax.axis_index("s") * 2 + lax.axis_index("c")
        pltpu.sync_copy(
            xT_hbm.at[pl.ds(0, NB_T), pl.ds(wid * 128, 128)], idxT
        )

        iota = lax.iota(jnp.int32, L)
        rot = [(iota + k) & 15 for k in range(L)]

        def fire_gather(t, b):
            pltpu.async_copy(z_hbm.at[idxT.at[t]], G.at[b], gsem.at[b])

        fire_gather(0, 0)

        @pl.loop(0, NB_T, step=2)
        def outer(t0):
            for b in range(2):
                t = t0 + b
                b1 = (b + 1) % 2

                @pl.when(t + 1 < NB_T)
                def _():
                    fire_gather(t + 1, b1)

                pltpu.make_async_copy(
                    z_hbm.at[pl.ds(0, 128)], G.at[b], gsem.at[b]
                ).wait()

                @pl.when(t >= 2)
                def _():
                    pltpu.make_async_copy(
                        O.at[b],
                        o5_hbm.at[0, pl.ds(0, 8), 0],
                        wsem.at[b],
                    ).wait()

                @plsc.parallel_loop(0, 128, L, unroll=4)
                def _j(j0):
                    bc_j0 = jnp.broadcast_to(j0, (L,))
                    for c0 in range(0, D, L):
                        colv = c0 + iota
                        c8v = colv >> 3
                        c1v = colv & 7
                        for k in range(L):
                            rowv = bc_j0 + rot[k]
                            v = plsc.load_gather(G.at[b], [rowv, colv])
                            plsc.store_scatter(
                                O.at[b], [c8v, c1v, rowv], v * SCALE
                            )

                pltpu.async_copy(
                    O.at[b], o5_hbm.at[t, pl.ds(0, 8), wid], wsem.at[b]
                )

        for b in range(2):
            pltpu.make_async_copy(
                O.at[b], o5_hbm.at[0, pl.ds(0, 8), 0], wsem.at[b]
            ).wait()

    return stage2


def kernel(x, table):
    nb, nt = x.shape
    z = _make_stage1()(table.T)
    o5 = _make_stage2(nt)(x.T, jnp.reshape(z, (V, D)))
    return jnp.transpose(o5, (2, 4, 0, 1, 3)).reshape(nb, nt, D)


# speedup vs baseline: 2.0639x; 1.0021x over previous
import functools
import math

import jax
import jax.numpy as jnp
from jax import lax
from jax.experimental import pallas as pl
from jax.experimental.pallas import tpu as pltpu
from jax.experimental.pallas import tpu_sc as plsc

V = 1000000
D = 64
SCALE = math.sqrt(D)
L = 16
NW = 32

FULL_COLS = V // 128
REM = V - FULL_COLS * 128


def _make_stage1():
    mesh = plsc.VectorSubcoreMesh(core_axis_name="c", subcore_axis_name="s")

    @functools.partial(
        pl.kernel,
        mesh=mesh,
        out_type=jax.ShapeDtypeStruct((V // 2, 128), jnp.float32),
        scratch_types=[
            pltpu.VMEM((2, 64, 256), jnp.float32),
            pltpu.VMEM((2, 128, 128), jnp.float32),
            pltpu.VMEM((64, 64), jnp.float32),
            pltpu.VMEM((32, 128), jnp.float32),
            pltpu.SemaphoreType.DMA((2,)),
            pltpu.SemaphoreType.DMA((2,)),
        ],
        compiler_params=pltpu.CompilerParams(
            use_tc_tiling_on_sc=True, needs_layout_passes=False
        ),
    )
    def stage1(tabT_hbm, z_hbm, inb, outb, pin, pout, rsem, wsem):
        wid = lax.axis_index("s") * 2 + lax.axis_index("c")
        start = 244 * wid + 2 * jnp.minimum(wid, 2)
        n_pairs = jnp.where(wid < 2, 123, 122)

        iota = lax.iota(jnp.int32, L)
        rot = [(iota + k) & 15 for k in range(L)]

        def transpose_block(src, dst, n_m, m_base):
            @plsc.parallel_loop(0, n_m, L, unroll=2)
            def _t(m0):
                bc_m0 = jnp.broadcast_to(m0, (L,))
                for c0 in range(0, D, L):
                    crow = c0 + iota
                    for k in range(L):
                        colv = bc_m0 + ((iota + k) & 15)
                        orow = m_base + (colv >> 1)
                        ocol = ((colv & 1) << 6) + crow
                        v = plsc.load_gather(src, [crow, colv])
                        plsc.store_scatter(dst, [orow, ocol], v)

        def fire_read(i, b):
            tc = start + 2 * i
            pltpu.async_copy(
                tabT_hbm.at[pl.ds(0, D), pl.ds(tc * 128, 256)],
                inb.at[b],
                rsem.at[b],
            )

        fire_read(0, 0)

        @pl.loop(0, 124, step=2)
        def outer(i0):
            for b in range(2):
                i = i0 + b
                b1 = (b + 1) % 2

                @pl.when(i < n_pairs)
                def _():
                    @pl.when(i + 1 < n_pairs)
                    def _():
                        fire_read(i + 1, b1)

                    pltpu.make_async_copy(
                        tabT_hbm.at[pl.ds(0, D), pl.ds(0, 256)],
                        inb.at[b],
                        rsem.at[b],
                    ).wait()

                    @pl.when(i >= 2)
                    def _():
                        pltpu.make_async_copy(
                            outb.at[b],
                            z_hbm.at[pl.ds(0, 128)],
                            wsem.at[b],
                        ).wait()

                    transpose_block(inb.at[b], outb.at[b], 256, 0)

                    tc = start + 2 * i
                    pltpu.async_copy(
                        outb.at[b],
                        z_hbm.at[pl.ds(tc * 64, 128)],
                        wsem.at[b],
                    )

        @pl.when(wid == NW - 1)
        def _():
            pltpu.sync_copy(
                tabT_hbm.at[pl.ds(0, D), pl.ds(FULL_COLS * 128, REM)], pin
            )

            transpose_block(pin, pout, REM, 0)

            pltpu.sync_copy(pout, z_hbm.at[pl.ds(FULL_COLS * 64, REM // 2)])

        for b in range(2):
            @pl.when(n_pairs >= 2 - b)
            def _():
                pltpu.make_async_copy(
                    outb.at[b], z_hbm.at[pl.ds(0, 128)], wsem.at[b]
                ).wait()

    return stage1


def _make_stage2(NB_T):
    mesh = plsc.VectorSubcoreMesh(core_axis_name="c", subcore_axis_name="s")

    @functools.partial(
        pl.kernel,
        mesh=mesh,
        out_type=jax.ShapeDtypeStruct((NB_T, 8, 32, 8, 128), jnp.float32),
        scratch_types=[
            pltpu.VMEM((NB_T, 128), jnp.int32),
            pltpu.VMEM((2, 128, D), jnp.float32),
            pltpu.VMEM((2, 8, 8, 128), jnp.float32),
            pltpu.SemaphoreType.DMA((2,)),
            pltpu.SemaphoreType.DMA((2,)),
        ],
        compiler_params=pltpu.CompilerParams(
            use_tc_tiling_on_sc=False, needs_layout_passes=False
        ),
    )
    def stage2(xT_hbm, z_hbm, o5_hbm, idxT, G, O, gsem, wsem):
        wid = lax.axis_index("s") * 2 + lax.axis_index("c")
        pltpu.sync_copy(
            xT_hbm.at[pl.ds(0, NB_T), pl.ds(wid * 128, 128)], idxT
        )

        iota = lax.iota(jnp.int32, L)
        rot = [(iota + k) & 15 for k in range(L)]

        def fire_gather(t, b):
            pltpu.async_copy(z_hbm.at[idxT.at[t]], G.at[b], gsem.at[b])

        fire_gather(0, 0)

        @pl.loop(0, NB_T, step=2)
        def outer(t0):
            for b in range(2):
                t = t0 + b
                b1 = (b + 1) % 2

                @pl.when(t + 1 < NB_T)
                def _():
                    fire_gather(t + 1, b1)

                pltpu.make_async_copy(
                    z_hbm.at[pl.ds(0, 128)], G.at[b], gsem.at[b]
                ).wait()

                @pl.when(t >= 2)
                def _():
                    pltpu.make_async_copy(
                        O.at[b],
                        o5_hbm.at[0, pl.ds(0, 8), 0],
                        wsem.at[b],
                    ).wait()

                @plsc.parallel_loop(0, 128, L, unroll=4)
                def _j(j0):
                    bc_j0 = jnp.broadcast_to(j0, (L,))
                    for c0 in range(0, D, L):
                        colv = c0 + iota
                        c8v = colv >> 3
                        c1v = colv & 7
                        for k in range(L):
                            rowv = bc_j0 + rot[k]
                            v = plsc.load_gather(G.at[b], [rowv, colv])
                            plsc.store_scatter(
                                O.at[b], [c8v, c1v, rowv], v * SCALE
                            )

                pltpu.async_copy(
                    O.at[b], o5_hbm.at[t, pl.ds(0, 8), wid], wsem.at[b]
                )

        for b in range(2):
            pltpu.make_async_copy(
                O.at[b], o5_hbm.at[0, pl.ds(0, 8), 0], wsem.at[b]
            ).wait()

    return stage2


def kernel(x, table):
    nb, nt = x.shape
    z = _make_stage1()(table.T)
    o5 = _make_stage2(nt)(x.T, jnp.reshape(z, (V, D)))
    return jnp.transpose(o5, (2, 4, 0, 1, 3)).reshape(nb, nt, D)


# speedup vs baseline: 3.2409x; 1.5703x over previous
import functools
import math

import jax
import jax.numpy as jnp
from jax import lax
from jax.experimental import pallas as pl
from jax.experimental.pallas import tpu as pltpu
from jax.experimental.pallas import tpu_sc as plsc

V = 1000000
D = 64
SCALE = math.sqrt(D)
L = 16
NW = 32

FULL_COLS = V // 128
REM = V - FULL_COLS * 128


def _make_stage1():
    mesh = plsc.VectorSubcoreMesh(core_axis_name="c", subcore_axis_name="s")

    @functools.partial(
        pl.kernel,
        mesh=mesh,
        out_type=jax.ShapeDtypeStruct((V // 2, 128), jnp.float32),
        scratch_types=[
            pltpu.VMEM((2, 64, 128), jnp.float32),
            pltpu.VMEM((2, 64, 128), jnp.float32),
            pltpu.VMEM((64, 64), jnp.float32),
            pltpu.VMEM((32, 128), jnp.float32),
            pltpu.SemaphoreType.DMA((2,)),
            pltpu.SemaphoreType.DMA((2,)),
        ],
        compiler_params=pltpu.CompilerParams(
            use_tc_tiling_on_sc=True, needs_layout_passes=False
        ),
    )
    def stage1(tabT_hbm, z_hbm, inb, outb, pin, pout, rsem, wsem):
        wid = lax.axis_index("s") * 2 + lax.axis_index("c")
        start = 244 * wid + jnp.minimum(wid, 4)
        n_cols = jnp.where(wid < 4, 245, 244)

        iota = lax.iota(jnp.int32, L)
        rot = [(iota + k) & 15 for k in range(L)]

        def transpose_block(src, dst, n_m, m_base):
            @plsc.parallel_loop(0, n_m, L, unroll=4)
            def _t(m0):
                bc_m0 = jnp.broadcast_to(m0, (L,))
                for c0 in range(0, D, L):
                    crow = c0 + iota
                    for k in range(L):
                        colv = bc_m0 + ((iota + k) & 15)
                        orow = m_base + (colv >> 1)
                        ocol = ((colv & 1) << 6) + crow
                        v = plsc.load_gather(src, [crow, colv])
                        plsc.store_scatter(dst, [orow, ocol], v)

        def fire_read(i, b):
            tc = start + i
            pltpu.async_copy(
                tabT_hbm.at[pl.ds(0, D), pl.ds(tc * 128, 128)],
                inb.at[b],
                rsem.at[b],
            )

        fire_read(0, 0)

        @pl.loop(0, 246, step=2)
        def outer(i0):
            for b in range(2):
                i = i0 + b
                b1 = (b + 1) % 2

                @pl.when(i < n_cols)
                def _():
                    @pl.when(i + 1 < n_cols)
                    def _():
                        fire_read(i + 1, b1)

                    pltpu.make_async_copy(
                        tabT_hbm.at[pl.ds(0, D), pl.ds(0, 128)],
                        inb.at[b],
                        rsem.at[b],
                    ).wait()

                    @pl.when(i >= 2)
                    def _():
                        pltpu.make_async_copy(
                            outb.at[b],
                            z_hbm.at[pl.ds(0, 64)],
                            wsem.at[b],
                        ).wait()

                    transpose_block(inb.at[b], outb.at[b], 128, 0)

                    tc = start + i
                    pltpu.async_copy(
                        outb.at[b],
                        z_hbm.at[pl.ds(tc * 64, 64)],
                        wsem.at[b],
                    )

        @pl.when(wid == NW - 1)
        def _():
            pltpu.sync_copy(
                tabT_hbm.at[pl.ds(0, D), pl.ds(FULL_COLS * 128, REM)], pin
            )

            transpose_block(pin, pout, REM, 0)

            pltpu.sync_copy(pout, z_hbm.at[pl.ds(FULL_COLS * 64, REM // 2)])

        for b in range(2):
            @pl.when(n_cols >= 2 - b)
            def _():
                pltpu.make_async_copy(
                    outb.at[b], z_hbm.at[pl.ds(0, 64)], wsem.at[b]
                ).wait()

    return stage1


def _make_stage2(NB_T):
    mesh = plsc.VectorSubcoreMesh(core_axis_name="c", subcore_axis_name="s")

    @functools.partial(
        pl.kernel,
        mesh=mesh,
        out_type=jax.ShapeDtypeStruct((NB_T, 8, 32, 8, 128), jnp.float32),
        scratch_types=[
            pltpu.VMEM((NB_T, 128), jnp.int32),
            pltpu.VMEM((2, 128, D), jnp.float32),
            pltpu.VMEM((2, 8, 8, 128), jnp.float32),
            pltpu.SemaphoreType.DMA((2,)),
            pltpu.SemaphoreType.DMA((2,)),
        ],
        compiler_params=pltpu.CompilerParams(
            use_tc_tiling_on_sc=False, needs_layout_passes=False
        ),
    )
    def stage2(xT_hbm, z_hbm, o5_hbm, idxT, G, O, gsem, wsem):
        wid = lax.axis_index("s") * 2 + lax.axis_index("c")
        pltpu.sync_copy(
            xT_hbm.at[pl.ds(0, NB_T), pl.ds(wid * 128, 128)], idxT
        )

        iota = lax.iota(jnp.int32, L)
        rot = [(iota + k) & 15 for k in range(L)]

        def fire_gather(t, b):
            pltpu.async_copy(z_hbm.at[idxT.at[t]], G.at[b], gsem.at[b])

        fire_gather(0, 0)

        @pl.loop(0, NB_T, step=2)
        def outer(t0):
            for b in range(2):
                t = t0 + b
                b1 = (b + 1) % 2

                @pl.when(t + 1 < NB_T)
                def _():
                    fire_gather(t + 1, b1)

                pltpu.make_async_copy(
                    z_hbm.at[pl.ds(0, 128)], G.at[b], gsem.at[b]
                ).wait()

                @pl.when(t >= 2)
                def _():
                    pltpu.make_async_copy(
                        O.at[b],
                        o5_hbm.at[0, pl.ds(0, 8), 0],
                        wsem.at[b],
                    ).wait()

                @plsc.parallel_loop(0, 128, L, unroll=4)
                def _j(j0):
                    bc_j0 = jnp.broadcast_to(j0, (L,))
                    for c0 in range(0, D, L):
                        colv = c0 + iota
                        c8v = colv >> 3
                        c1v = colv & 7
                        for k in range(L):
                            rowv = bc_j0 + rot[k]
                            v = plsc.load_gather(G.at[b], [rowv, colv])
                            plsc.store_scatter(
                                O.at[b], [c8v, c1v, rowv], v * SCALE
                            )

                pltpu.async_copy(
                    O.at[b], o5_hbm.at[t, pl.ds(0, 8), wid], wsem.at[b]
                )

        for b in range(2):
            pltpu.make_async_copy(
                O.at[b], o5_hbm.at[0, pl.ds(0, 8), 0], wsem.at[b]
            ).wait()

    return stage2


def kernel(x, table):
    nb, nt = x.shape
    z = _make_stage1()(table.T)
    o5 = _make_stage2(nt)(x.T, jnp.reshape(z, (V, D)))
    return jnp.transpose(o5, (2, 4, 0, 1, 3)).reshape(nb, nt, D)
